# Initial kernel scaffold; baseline (speedup 1.0000x reference)
import jax
import jax.numpy as jnp
from jax.experimental import pallas as pl


def kernel(index_list, emb, W1, b1, W2, b2):
    raise NotImplementedError("write your pallas kernel here")


# speedup vs baseline: 6.6891x; 6.6891x over previous
import functools

import jax
import jax.numpy as jnp
from jax import lax
from jax.experimental import pallas as pl
from jax.experimental.pallas import tpu as pltpu
from jax.experimental.pallas import tpu_sc as plsc

VOCAB = 100000
D = 128
B = 4096
L = 50
H1 = 100
H2 = 50

NC = 2
NS = 16
NW = NC * NS
BPW = B // NW
ROWS_PER_CHUNK = 128
NCHUNK = (BPW * L) // ROWS_PER_CHUNK

_MESH = plsc.VectorSubcoreMesh(core_axis_name="c", subcore_axis_name="s")


@functools.partial(
    pl.kernel,
    mesh=_MESH,
    out_type=jax.ShapeDtypeStruct((B, D), jnp.float32),
    scratch_types=[
        pltpu.VMEM((NCHUNK, ROWS_PER_CHUNK), jnp.int32),
        pltpu.VMEM((NCHUNK, ROWS_PER_CHUNK), jnp.int32),
        pltpu.VMEM((ROWS_PER_CHUNK, D), jnp.float32),
        pltpu.VMEM_SHARED((NS * BPW, D), jnp.float32),
        pltpu.SemaphoreType.DMA,
    ],
)
def _sc_pool(emb_hbm, idx_hbm, slot_hbm, out_hbm, idx_v, slot_v, rows_v,
             acc_sh, sem):
    sid = lax.axis_index("s")
    wid = sid * NC + lax.axis_index("c")

    pltpu.sync_copy(idx_hbm.at[wid], idx_v)
    pltpu.sync_copy(slot_hbm.at[wid], slot_v)

    zeros16 = jnp.zeros((16,), jnp.float32)

    @pl.loop(0, ROWS_PER_CHUNK)
    def _zero_row(r):
        @pl.loop(0, D, step=16)
        def _zero_col(j):
            rows_v[r, pl.ds(j, 16)] = zeros16

    pltpu.sync_copy(rows_v, acc_sh.at[pl.ds(sid * BPW, BPW)])

    @pl.loop(0, NCHUNK)
    def _chunk(c):
        pltpu.async_copy(emb_hbm.at[idx_v.at[c]], rows_v, sem).wait()
        pltpu.sync_copy(rows_v, acc_sh.at[slot_v.at[c]], add=True)

    pltpu.sync_copy(acc_sh.at[pl.ds(sid * BPW, BPW)],
                    out_hbm.at[pl.ds(wid * BPW, BPW)])


def _mlp_body(x_ref, w1_ref, b1_ref, w2_ref, b2_ref, o_ref):
    x = x_ref[...] * jnp.float32(1.0 / L)
    h = jnp.dot(x, w1_ref[...], preferred_element_type=jnp.float32)
    h = jnp.maximum(h + b1_ref[...], 0.0)
    logits = jnp.dot(h, w2_ref[...], preferred_element_type=jnp.float32)
    logits = logits + b2_ref[...]
    m = jnp.max(logits, axis=1, keepdims=True)
    s = logits - m
    lse = jnp.log(jnp.sum(jnp.exp(s), axis=1, keepdims=True))
    o_ref[...] = s - lse


_MLP_BLOCK = 1024


def _mlp(pooled, W1, b1, W2, b2):
    grid = (B // _MLP_BLOCK,)
    return pl.pallas_call(
        _mlp_body,
        grid=grid,
        in_specs=[
            pl.BlockSpec((_MLP_BLOCK, D), lambda i: (i, 0)),
            pl.BlockSpec((D, H1), lambda i: (0, 0)),
            pl.BlockSpec((1, H1), lambda i: (0, 0)),
            pl.BlockSpec((H1, H2), lambda i: (0, 0)),
            pl.BlockSpec((1, H2), lambda i: (0, 0)),
        ],
        out_specs=pl.BlockSpec((_MLP_BLOCK, H2), lambda i: (i, 0)),
        out_shape=jax.ShapeDtypeStruct((B, H2), jnp.float32),
    )(pooled, W1, b1, W2, b2)


def kernel(index_list, emb, W1, b1, W2, b2):
    idx = index_list.astype(jnp.int32).reshape(NW, NCHUNK, ROWS_PER_CHUNK)
    b = jnp.arange(B * L, dtype=jnp.int32) // L
    slots = (((b // BPW) // NC) * BPW + b % BPW).reshape(
        NW, NCHUNK, ROWS_PER_CHUNK)
    pooled = _sc_pool(emb, idx, slots)
    return _mlp(pooled, W1, b1.reshape(1, H1), W2, b2.reshape(1, H2))


# speedup vs baseline: 9.1457x; 1.3673x over previous
import functools

import jax
import jax.numpy as jnp
from jax import lax
from jax.experimental import pallas as pl
from jax.experimental.pallas import tpu as pltpu
from jax.experimental.pallas import tpu_sc as plsc

VOCAB = 100000
D = 128
B = 4096
L = 50
H1 = 100
H2 = 50

NC = 2
NS = 16
NW = NC * NS
BPW = B // NW
ROWS_PER_CHUNK = 128
NCHUNK = (BPW * L) // ROWS_PER_CHUNK

_MESH = plsc.VectorSubcoreMesh(core_axis_name="c", subcore_axis_name="s")


@functools.partial(
    pl.kernel,
    mesh=_MESH,
    out_type=jax.ShapeDtypeStruct((B, D), jnp.float32),
    scratch_types=[
        pltpu.VMEM((NCHUNK, ROWS_PER_CHUNK), jnp.int32),
        pltpu.VMEM((NCHUNK, ROWS_PER_CHUNK), jnp.int32),
        pltpu.VMEM((ROWS_PER_CHUNK, D), jnp.float32),
        pltpu.VMEM((ROWS_PER_CHUNK, D), jnp.float32),
        pltpu.VMEM_SHARED((NS * BPW, D), jnp.float32),
        pltpu.SemaphoreType.DMA,
        pltpu.SemaphoreType.DMA,
    ],
)
def _sc_pool(emb_hbm, idx_hbm, slot_hbm, out_hbm, idx_v, slot_v, rows_a,
             rows_b, acc_sh, sem_a, sem_b):
    sid = lax.axis_index("s")
    wid = sid * NC + lax.axis_index("c")

    pltpu.sync_copy(idx_hbm.at[wid], idx_v)
    pltpu.sync_copy(slot_hbm.at[wid], slot_v)

    zeros16 = jnp.zeros((16,), jnp.float32)

    @pl.loop(0, ROWS_PER_CHUNK)
    def _zero_row(r):
        @pl.loop(0, D, step=16)
        def _zero_col(j):
            rows_a[r, pl.ds(j, 16)] = zeros16

    pltpu.sync_copy(rows_a, acc_sh.at[pl.ds(sid * BPW, BPW)])

    pltpu.async_copy(emb_hbm.at[idx_v.at[0]], rows_a, sem_a)
    pltpu.async_copy(emb_hbm.at[idx_v.at[1]], rows_b, sem_b)

    @pl.loop(0, NCHUNK, step=2)
    def _chunk(c):
        pltpu.make_async_copy(emb_hbm.at[idx_v.at[c]], rows_a, sem_a).wait()
        pltpu.sync_copy(rows_a, acc_sh.at[slot_v.at[c]], add=True)

        @pl.when(c + 2 < NCHUNK)
        def _():
            pltpu.async_copy(emb_hbm.at[idx_v.at[c + 2]], rows_a, sem_a)

        pltpu.make_async_copy(emb_hbm.at[idx_v.at[c]], rows_b, sem_b).wait()
        pltpu.sync_copy(rows_b, acc_sh.at[slot_v.at[c + 1]], add=True)

        @pl.when(c + 3 < NCHUNK)
        def _():
            pltpu.async_copy(emb_hbm.at[idx_v.at[c + 3]], rows_b, sem_b)

    pltpu.sync_copy(acc_sh.at[pl.ds(sid * BPW, BPW)],
                    out_hbm.at[pl.ds(wid * BPW, BPW)])


def _mlp_body(x_ref, w1_ref, b1_ref, w2_ref, b2_ref, o_ref):
    x = x_ref[...] * jnp.float32(1.0 / L)
    h = jnp.dot(x, w1_ref[...], preferred_element_type=jnp.float32)
    h = jnp.maximum(h + b1_ref[...], 0.0)
    logits = jnp.dot(h, w2_ref[...], preferred_element_type=jnp.float32)
    logits = logits + b2_ref[...]
    m = jnp.max(logits, axis=1, keepdims=True)
    s = logits - m
    lse = jnp.log(jnp.sum(jnp.exp(s), axis=1, keepdims=True))
    o_ref[...] = s - lse


_MLP_BLOCK = 1024


def _mlp(pooled, W1, b1, W2, b2):
    grid = (B // _MLP_BLOCK,)
    return pl.pallas_call(
        _mlp_body,
        grid=grid,
        in_specs=[
            pl.BlockSpec((_MLP_BLOCK, D), lambda i: (i, 0)),
            pl.BlockSpec((D, H1), lambda i: (0, 0)),
            pl.BlockSpec((1, H1), lambda i: (0, 0)),
            pl.BlockSpec((H1, H2), lambda i: (0, 0)),
            pl.BlockSpec((1, H2), lambda i: (0, 0)),
        ],
        out_specs=pl.BlockSpec((_MLP_BLOCK, H2), lambda i: (i, 0)),
        out_shape=jax.ShapeDtypeStruct((B, H2), jnp.float32),
    )(pooled, W1, b1, W2, b2)


def kernel(index_list, emb, W1, b1, W2, b2):
    idx = index_list.astype(jnp.int32).reshape(NW, NCHUNK, ROWS_PER_CHUNK)
    b = jnp.arange(B * L, dtype=jnp.int32) // L
    slots = (((b // BPW) // NC) * BPW + b % BPW).reshape(
        NW, NCHUNK, ROWS_PER_CHUNK)
    pooled = _sc_pool(emb, idx, slots)
    return _mlp(pooled, W1, b1.reshape(1, H1), W2, b2.reshape(1, H2))


# speedup vs baseline: 9.8647x; 1.0786x over previous
import functools

import jax
import jax.numpy as jnp
from jax import lax
from jax.experimental import pallas as pl
from jax.experimental.pallas import tpu as pltpu
from jax.experimental.pallas import tpu_sc as plsc

VOCAB = 100000
D = 128
B = 4096
L = 50
H1 = 100
H2 = 50

NC = 2
NS = 16
NW = NC * NS
BPW = B // NW
ROWS_PER_CHUNK = 128
NCHUNK = (BPW * L) // ROWS_PER_CHUNK

_MESH = plsc.VectorSubcoreMesh(core_axis_name="c", subcore_axis_name="s")


@functools.partial(
    pl.kernel,
    mesh=_MESH,
    out_type=jax.ShapeDtypeStruct((B, D), jnp.float32),
    scratch_types=[
        pltpu.VMEM((NCHUNK, ROWS_PER_CHUNK), jnp.int32),
        pltpu.VMEM((NCHUNK, ROWS_PER_CHUNK), jnp.int32),
        pltpu.VMEM((ROWS_PER_CHUNK, D), jnp.float32),
        pltpu.VMEM((ROWS_PER_CHUNK, D), jnp.float32),
        pltpu.VMEM((ROWS_PER_CHUNK, D), jnp.float32),
        pltpu.VMEM((ROWS_PER_CHUNK, D), jnp.float32),
        pltpu.VMEM((ROWS_PER_CHUNK, D), jnp.float32),
        pltpu.VMEM_SHARED((NS * BPW, D), jnp.float32),
        pltpu.SemaphoreType.DMA,
        pltpu.SemaphoreType.DMA,
        pltpu.SemaphoreType.DMA,
        pltpu.SemaphoreType.DMA,
        pltpu.SemaphoreType.DMA,
    ],
)
def _sc_pool(emb_hbm, idx_hbm, slot_hbm, out_hbm, idx_v, slot_v, rows_0,
             rows_1, rows_2, rows_3, rows_4, acc_sh, sem_0, sem_1, sem_2,
             sem_3, sem_4):
    sid = lax.axis_index("s")
    wid = sid * NC + lax.axis_index("c")

    pltpu.sync_copy(idx_hbm.at[wid], idx_v)
    pltpu.sync_copy(slot_hbm.at[wid], slot_v)

    zeros16 = jnp.zeros((16,), jnp.float32)

    @pl.loop(0, ROWS_PER_CHUNK)
    def _zero_row(r):
        @pl.loop(0, D, step=16)
        def _zero_col(j):
            rows_0[r, pl.ds(j, 16)] = zeros16

    pltpu.sync_copy(rows_0, acc_sh.at[pl.ds(sid * BPW, BPW)])

    bufs = (rows_0, rows_1, rows_2, rows_3, rows_4)
    sems = (sem_0, sem_1, sem_2, sem_3, sem_4)
    nbuf = len(bufs)

    for k in range(nbuf):
        pltpu.async_copy(emb_hbm.at[idx_v.at[k]], bufs[k], sems[k])

    @pl.loop(0, NCHUNK, step=nbuf)
    def _chunk(c):
        for k in range(nbuf):
            pltpu.make_async_copy(
                emb_hbm.at[idx_v.at[0]], bufs[k], sems[k]).wait()
            pltpu.sync_copy(bufs[k], acc_sh.at[slot_v.at[c + k]], add=True)

            @pl.when(c + k + nbuf < NCHUNK)
            def _(k=k):
                pltpu.async_copy(
                    emb_hbm.at[idx_v.at[c + k + nbuf]], bufs[k], sems[k])

    pltpu.sync_copy(acc_sh.at[pl.ds(sid * BPW, BPW)],
                    out_hbm.at[pl.ds(wid * BPW, BPW)])


def _mlp_body(x_ref, w1_ref, b1_ref, w2_ref, b2_ref, o_ref):
    x = x_ref[...] * jnp.float32(1.0 / L)
    h = jnp.dot(x, w1_ref[...], preferred_element_type=jnp.float32)
    h = jnp.maximum(h + b1_ref[...], 0.0)
    logits = jnp.dot(h, w2_ref[...], preferred_element_type=jnp.float32)
    logits = logits + b2_ref[...]
    m = jnp.max(logits, axis=1, keepdims=True)
    s = logits - m
    lse = jnp.log(jnp.sum(jnp.exp(s), axis=1, keepdims=True))
    o_ref[...] = s - lse


_MLP_BLOCK = 1024


def _mlp(pooled, W1, b1, W2, b2):
    grid = (B // _MLP_BLOCK,)
    return pl.pallas_call(
        _mlp_body,
        grid=grid,
        in_specs=[
            pl.BlockSpec((_MLP_BLOCK, D), lambda i: (i, 0)),
            pl.BlockSpec((D, H1), lambda i: (0, 0)),
            pl.BlockSpec((1, H1), lambda i: (0, 0)),
            pl.BlockSpec((H1, H2), lambda i: (0, 0)),
            pl.BlockSpec((1, H2), lambda i: (0, 0)),
        ],
        out_specs=pl.BlockSpec((_MLP_BLOCK, H2), lambda i: (i, 0)),
        out_shape=jax.ShapeDtypeStruct((B, H2), jnp.float32),
    )(pooled, W1, b1, W2, b2)


def kernel(index_list, emb, W1, b1, W2, b2):
    idx = index_list.astype(jnp.int32).reshape(NW, NCHUNK, ROWS_PER_CHUNK)
    b = jnp.arange(B * L, dtype=jnp.int32) // L
    slots = (((b // BPW) // NC) * BPW + b % BPW).reshape(
        NW, NCHUNK, ROWS_PER_CHUNK)
    pooled = _sc_pool(emb, idx, slots)
    return _mlp(pooled, W1, b1.reshape(1, H1), W2, b2.reshape(1, H2))
